# Initial kernel scaffold; baseline (speedup 1.0000x reference)
import jax
import jax.numpy as jnp
from jax.experimental import pallas as pl


def kernel(x, inds_b, inds_x, inds_y, W1, b1, W2, b2):
    raise NotImplementedError("write your pallas kernel here")


# speedup vs baseline: 2.1518x; 2.1518x over previous
import jax
import jax.numpy as jnp
from jax.experimental import pallas as pl


def _mlp_body(x_ref, w1_ref, b1_ref, w2_ref, b2_ref, o_ref):
    h = jnp.maximum(x_ref[...] @ w1_ref[...] + b1_ref[...], 0.0)
    r = jnp.maximum(h @ w2_ref[...] + b2_ref[...], 0.0)
    o_ref[...] = r


def kernel(x, inds_b, inds_x, inds_y, W1, b1, W2, b2):
    N, D = x.shape
    G = 512
    B = 4
    BLK = 10000
    reg_evi = pl.pallas_call(
        _mlp_body,
        grid=(N // BLK,),
        in_specs=[
            pl.BlockSpec((BLK, D), lambda i: (i, 0)),
            pl.BlockSpec((D, 32), lambda i: (0, 0)),
            pl.BlockSpec((1, 32), lambda i: (0, 0)),
            pl.BlockSpec((32, 2), lambda i: (0, 0)),
            pl.BlockSpec((1, 2), lambda i: (0, 0)),
        ],
        out_specs=pl.BlockSpec((BLK, 2), lambda i: (i, 0)),
        out_shape=jax.ShapeDtypeStruct((N, 2), jnp.float32),
    )(x, W1, b1.reshape(1, 32), W2, b2.reshape(1, 2))
    cell = (inds_b * G + inds_x) * G + inds_y
    order = jnp.arange(N, dtype=jnp.int32)
    winner = jnp.full((B * G * G,), -1, dtype=jnp.int32).at[cell].max(order)
    vals = jnp.where((winner >= 0)[:, None], reg_evi[winner.clip(0)], 0.0)
    return vals.reshape(B, G, G, 2)


# speedup vs baseline: 2.2036x; 1.0240x over previous
import functools

import jax
import jax.numpy as jnp
from jax import lax
from jax.experimental import pallas as pl
from jax.experimental.pallas import tpu as pltpu
from jax.experimental.pallas import tpu_sc as plsc

N = 1000000
D = 64
G = 512
BATCH = 4
NCELL = BATCH * G * G
NOWN = 33
OWNER_SHIFT = 15
CPB = 32768
NW = 32
LANES = 16
VPW = 1960
VT = NW * VPW
NPAD = VT * LANES
NROW = 62500
RPW = VPW // 8
STMP = NPAD + NOWN * 8
STMP_ALLOC = STMP + 2048
MLP_BLK = 12800
REG_ROWS = 79 * MLP_BLK
CH = 2048


def _mlp_body(x_ref, w1_ref, b1_ref, w2_ref, b2_ref, o_ref):
    pid = pl.program_id(0)
    h = jnp.maximum(x_ref[...] @ w1_ref[...] + b1_ref[...], 0.0)
    r = jnp.maximum(h @ w2_ref[...] + b2_ref[...], 0.0)
    rows = pid * MLP_BLK + lax.broadcasted_iota(jnp.int32, (MLP_BLK, 1), 0)
    o_ref[...] = jnp.where(rows < N, r, 0.0)


def _mlp_call(x, W1, b1, W2, b2):
    return pl.pallas_call(
        _mlp_body,
        grid=(REG_ROWS // MLP_BLK,),
        in_specs=[
            pl.BlockSpec((MLP_BLK, D), lambda i: (i, 0)),
            pl.BlockSpec((D, 32), lambda i: (0, 0)),
            pl.BlockSpec((1, 32), lambda i: (0, 0)),
            pl.BlockSpec((32, 2), lambda i: (0, 0)),
            pl.BlockSpec((1, 2), lambda i: (0, 0)),
        ],
        out_specs=pl.BlockSpec((MLP_BLK, 2), lambda i: (i, 0)),
        out_shape=jax.ShapeDtypeStruct((REG_ROWS, 2), jnp.float32),
    )(x, W1, b1.reshape(1, 32), W2, b2.reshape(1, 2))


def _cells_body(b_ref, x_ref, y_ref, c_ref, h_ref):
    i = pl.program_id(0)
    rows = i * VPW + lax.broadcasted_iota(jnp.int32, (VPW, LANES), 0)
    valid = rows < NROW
    cell = (b_ref[...] * G + x_ref[...]) * G + y_ref[...]
    cell = jnp.where(valid, cell, NCELL)
    c_ref[...] = cell
    owner = lax.shift_right_logical(cell, OWNER_SHIFT)
    counts = [jnp.sum((owner == o).astype(jnp.int32), axis=0) for o in range(NOWN)]
    h_ref[...] = jnp.stack(counts, axis=0)[None]


def _cells_call(ib2, ix2, iy2):
    spec_in = pl.BlockSpec((VPW, LANES), lambda i: (i, 0))
    return pl.pallas_call(
        _cells_body,
        grid=(NW,),
        in_specs=[spec_in, spec_in, spec_in],
        out_specs=[
            pl.BlockSpec((VPW, LANES), lambda i: (i, 0)),
            pl.BlockSpec((1, NOWN, LANES), lambda i: (i, 0, 0)),
        ],
        out_shape=[
            jax.ShapeDtypeStruct((VT, LANES), jnp.int32),
            jax.ShapeDtypeStruct((NW, NOWN, LANES), jnp.int32),
        ],
    )(ib2, ix2, iy2)


def _tri_strict(n):
    r = lax.broadcasted_iota(jnp.int32, (n, n), 0)
    c = lax.broadcasted_iota(jnp.int32, (n, n), 1)
    return (r > c).astype(jnp.float32)


def _off_body(h_ref, offt_ref, sb_ref):
    H = h_ref[...]
    Hf = H.astype(jnp.float32)
    sum_l = jnp.sum(H, axis=2)
    hi = jax.lax.Precision.HIGHEST
    s1ex = jnp.dot(_tri_strict(NW), sum_l.astype(jnp.float32),
                   precision=hi).astype(jnp.int32)
    u = _tri_strict(LANES).T
    s2ex = jnp.dot(Hf.reshape(NW * NOWN, LANES), u,
                   precision=hi).astype(jnp.int32).reshape(NW, NOWN, LANES)
    t = jnp.sum(sum_l, axis=0)
    pt = ((t + 7) // 8) * 8
    startx = jnp.dot(_tri_strict(NOWN), pt.astype(jnp.float32),
                     precision=hi).astype(jnp.int32)
    offt_ref[...] = startx[None, :, None] + s1ex[:, :, None] + s2ex
    start40 = jnp.pad(startx, (0, 40 - NOWN))
    t40 = jnp.pad(t, (0, 40 - NOWN))
    sb = jnp.stack([start40, t40], axis=0)
    sb_ref[...] = jnp.pad(sb, ((0, 6), (0, 0)))


def _off_call(Hout):
    return pl.pallas_call(
        _off_body,
        out_shape=[
            jax.ShapeDtypeStruct((NW, NOWN, LANES), jnp.int32),
            jax.ShapeDtypeStruct((8, 40), jnp.int32),
        ],
    )(Hout)


_MESH = dict(core_axis_name="c", subcore_axis_name="s")


def _permute_body(cells_hbm, offt_hbm, ctmp, itmp, cslab, offs, dbuf,
                  ivbuf, sem_c, sem_i):
    wid = lax.axis_index("s") * 2 + lax.axis_index("c")
    pltpu.sync_copy(cells_hbm.at[pl.ds(wid * (VPW * LANES), VPW * LANES)], cslab)
    pltpu.sync_copy(offt_hbm.at[wid], offs)
    iota = lax.iota(jnp.int32, LANES)
    ones = jnp.ones((LANES,), jnp.int32)
    base_i = wid * (VPW * LANES)

    def row_body(r, carry):
        for j in range(8):
            k = r * 8 + j
            c = cslab[pl.ds(k * 16, 16)]
            owner = lax.shift_right_logical(c, OWNER_SHIFT)
            tbl = owner * 16 + iota
            dest = plsc.load_gather(offs, [tbl])
            plsc.addupdate_scatter(offs, [tbl], ones)
            plsc.store_scatter(dbuf, [jnp.full((LANES,), r, jnp.int32),
                                      j * 16 + iota], dest)
            plsc.store_scatter(ivbuf, [jnp.full((LANES,), r, jnp.int32),
                                       j * 16 + iota], base_i + k * 16 + iota)
        pltpu.async_copy(cslab.at[pl.ds(r * 128, 128)], ctmp.at[dbuf.at[r]], sem_c)
        pltpu.async_copy(ivbuf.at[r], itmp.at[dbuf.at[r]], sem_i)
        return carry

    lax.fori_loop(0, RPW, row_body, 0)

    def drain_c(r, carry):
        pltpu.make_async_copy(cslab.at[pl.ds(0, 128)], ctmp.at[dbuf.at[0]], sem_c).wait()
        pltpu.make_async_copy(ivbuf.at[0], itmp.at[dbuf.at[0]], sem_i).wait()
        return carry

    lax.fori_loop(0, RPW, drain_c, 0)


def _permute_call(cells1, offT):
    mesh = plsc.VectorSubcoreMesh(**_MESH)
    f = pl.kernel(
        _permute_body,
        out_type=(
            jax.ShapeDtypeStruct((STMP_ALLOC,), jnp.int32),
            jax.ShapeDtypeStruct((STMP_ALLOC,), jnp.int32),
        ),
        mesh=mesh,
        compiler_params=pltpu.CompilerParams(needs_layout_passes=False),
        scratch_types=[
            pltpu.VMEM((VPW * LANES,), jnp.int32),
            pltpu.VMEM((NOWN * LANES,), jnp.int32),
            pltpu.VMEM((RPW, 128), jnp.int32),
            pltpu.VMEM((RPW, 128), jnp.int32),
            pltpu.SemaphoreType.DMA,
            pltpu.SemaphoreType.DMA,
        ],
    )
    return f(cells1, offT)


def _apply_body(ctmp, itmp, sb_hbm, reg0_hbm, reg1_hbm, out0_hbm, out1_hbm,
                W, vals0, vals1, cbuf, ibuf, sbv, sem):
    wid = lax.axis_index("s") * 2 + lax.axis_index("c")
    pltpu.sync_copy(sb_hbm, sbv)
    iota = lax.iota(jnp.int32, LANES)
    zeros16 = jnp.zeros((LANES,), jnp.int32)
    s0 = jnp.max(plsc.load_gather(sbv, [zeros16, jnp.full((LANES,), wid, jnp.int32)]))
    s1 = jnp.max(plsc.load_gather(sbv, [zeros16, jnp.full((LANES,), wid + 1, jnp.int32)]))
    cnt = jnp.max(plsc.load_gather(sbv, [zeros16 + 1, jnp.full((LANES,), wid, jnp.int32)]))
    lim = s0 + cnt

    def initw(v, carry):
        W[pl.ds(v * 16, 16)] = jnp.full((LANES,), -1, jnp.int32)
        return carry

    lax.fori_loop(0, CPB // 16, initw, 0)

    nchunks = (s1 - s0 + CH - 1) // CH

    def chunk_body(ch, carry):
        base = pl.multiple_of(s0 + ch * CH, 8)
        pltpu.sync_copy(ctmp.at[pl.ds(base, CH)], cbuf)
        pltpu.sync_copy(itmp.at[pl.ds(base, CH)], ibuf)

        def vbody(v, c2):
            c = cbuf[pl.ds(v * 16, 16)]
            ii = ibuf[pl.ds(v * 16, 16)]
            pos = base + v * 16 + iota
            valid = pos < lim
            local = jnp.bitwise_and(c, CPB - 1)
            w0 = plsc.load_gather(W, [local])

            def wcond(st):
                _, m = st
                return jnp.any(m)

            def wbody(st):
                _, m = st
                plsc.store_scatter(W, [local], ii, mask=m)
                w2 = plsc.load_gather(W, [local])
                return (w2, valid & (ii > w2))

            lax.while_loop(wcond, wbody, (w0, valid & (ii > w0)))
            return c2

        lax.fori_loop(0, CH // 16, vbody, 0)
        return carry

    lax.fori_loop(0, nchunks, chunk_body, 0)

    def fixw(v, carry):
        w = W[pl.ds(v * 16, 16)]
        W[pl.ds(v * 16, 16)] = jnp.where(w < 0, jnp.full((LANES,), N, jnp.int32), w)
        return carry

    lax.fori_loop(0, CPB // 16, fixw, 0)

    def gbody(g, carry):
        pltpu.async_copy(reg0_hbm.at[W.at[pl.ds(g * 128, 128)]],
                         vals0.at[pl.ds(g * 128, 128)], sem)
        pltpu.async_copy(reg1_hbm.at[W.at[pl.ds(g * 128, 128)]],
                         vals1.at[pl.ds(g * 128, 128)], sem)

        @pl.when(g >= 8)
        def _drain_one():
            pltpu.make_async_copy(reg0_hbm.at[W.at[pl.ds(0, 128)]],
                                  vals0.at[pl.ds(0, 128)], sem).wait()
            pltpu.make_async_copy(reg1_hbm.at[W.at[pl.ds(0, 128)]],
                                  vals1.at[pl.ds(0, 128)], sem).wait()

        return carry

    lax.fori_loop(0, CPB // 128, gbody, 0)

    def gdrain(g, carry):
        pltpu.make_async_copy(reg0_hbm.at[W.at[pl.ds(0, 128)]],
                              vals0.at[pl.ds(0, 128)], sem).wait()
        pltpu.make_async_copy(reg1_hbm.at[W.at[pl.ds(0, 128)]],
                              vals1.at[pl.ds(0, 128)], sem).wait()
        return carry

    lax.fori_loop(0, 8, gdrain, 0)
    pltpu.sync_copy(vals0, out0_hbm.at[pl.ds(wid * CPB, CPB)])
    pltpu.sync_copy(vals1, out1_hbm.at[pl.ds(wid * CPB, CPB)])


def _apply_call(ctmp, itmp, sb, reg0, reg1):
    mesh = plsc.VectorSubcoreMesh(**_MESH)
    f = pl.kernel(
        _apply_body,
        out_type=(
            jax.ShapeDtypeStruct((NCELL,), jnp.float32),
            jax.ShapeDtypeStruct((NCELL,), jnp.float32),
        ),
        mesh=mesh,
        compiler_params=pltpu.CompilerParams(needs_layout_passes=False),
        scratch_types=[
            pltpu.VMEM((CPB,), jnp.int32),
            pltpu.VMEM((CPB,), jnp.float32),
            pltpu.VMEM((CPB,), jnp.float32),
            pltpu.VMEM((CH,), jnp.int32),
            pltpu.VMEM((CH,), jnp.int32),
            pltpu.VMEM((8, 40), jnp.int32),
            pltpu.SemaphoreType.DMA,
        ],
    )
    return f(ctmp, itmp, sb, reg0, reg1)


def kernel(x, inds_b, inds_x, inds_y, W1, b1, W2, b2):
    reg = _mlp_call(x, W1, b1, W2, b2)
    reg0 = reg[:, 0]
    reg1 = reg[:, 1]
    ib2 = inds_b.reshape(NROW, LANES)
    ix2 = inds_x.reshape(NROW, LANES)
    iy2 = inds_y.reshape(NROW, LANES)
    cells2d, Hout = _cells_call(ib2, ix2, iy2)
    offT3, sb = _off_call(Hout)
    cells1 = cells2d.reshape(-1)
    offT = offT3.reshape(NW, NOWN * LANES)
    ctmp, itmp = _permute_call(cells1, offT)
    out0, out1 = _apply_call(ctmp, itmp, sb, reg0, reg1)
    return jnp.stack([out0, out1], axis=-1).reshape(BATCH, G, G, 2)
